# baseline (speedup 1.0000x reference)
import jax
import jax.numpy as jnp
from jax import lax
from jax.experimental import pallas as pl
from jax.experimental.pallas import tpu as pltpu

_NBUF = 3
_SL = 16


def _stream_body(img_hbm, w_ref, p_ref, out_hbm, mw_ref, buf, isem, osem):
    nch = img_hbm.shape[0] // _SL
    mw_ref[...] = jnp.where(p_ref[...], 0.0, w_ref[...])

    def in_copy(c, b):
        return pltpu.make_async_copy(
            img_hbm.at[pl.ds(c * _SL, _SL)], buf.at[b], isem.at[b])

    def out_copy(c, b):
        return pltpu.make_async_copy(
            buf.at[b], out_hbm.at[pl.ds(c * _SL, _SL)], osem.at[b])

    for b in range(_NBUF):
        in_copy(b, b).start()

    def step(c, carry):
        b = lax.rem(c, _NBUF)
        in_copy(c, b).wait()
        buf[b] = buf[b] * mw_ref[...][None]
        out_copy(c, b).start()

        nb = lax.rem(c + 1, _NBUF)

        @pl.when(jnp.logical_and(c >= _NBUF - 1, c + 1 < nch))
        def _():
            out_copy(c + 1 - _NBUF, nb).wait()
            in_copy(c + 1, nb).start()

        return carry

    lax.fori_loop(0, nch, step, 0)
    for c in range(nch - _NBUF, nch):
        out_copy(c, c % _NBUF).wait()


def kernel(image, weight, pruned):
    B, C, H, W = image.shape
    img = image.reshape(B * C, H, W)
    out = pl.pallas_call(
        _stream_body,
        in_specs=[
            pl.BlockSpec(memory_space=pl.ANY),
            pl.BlockSpec(memory_space=pltpu.VMEM),
            pl.BlockSpec(memory_space=pltpu.VMEM),
        ],
        out_specs=pl.BlockSpec(memory_space=pl.ANY),
        out_shape=jax.ShapeDtypeStruct((B * C, H, W), jnp.float32),
        scratch_shapes=[
            pltpu.VMEM((H, W), jnp.float32),
            pltpu.VMEM((_NBUF, _SL, H, W), jnp.float32),
            pltpu.SemaphoreType.DMA((_NBUF,)),
            pltpu.SemaphoreType.DMA((_NBUF,)),
        ],
        compiler_params=pltpu.CompilerParams(
            vmem_limit_bytes=62 * 1024 * 1024,
        ),
    )(img, weight, pruned)
    return out.reshape(1, B, C, H, W)


# speedup vs baseline: 1.1612x; 1.1612x over previous
import jax
import jax.numpy as jnp
from jax.experimental import pallas as pl
from jax.experimental.pallas import tpu as pltpu


def _apply_body(img_ref, w_ref, p_ref, out_ref, mw_ref):
    @pl.when(pl.program_id(0) == 0)
    def _():
        mw_ref[...] = jnp.where(p_ref[...], 0.0, w_ref[...])

    out_ref[...] = img_ref[...] * mw_ref[...][None, :, :]


def kernel(image, weight, pruned):
    B, C, H, W = image.shape
    img = image.reshape(B * C, H, W)
    K = 8
    out = pl.pallas_call(
        _apply_body,
        grid=(B * C // K,),
        in_specs=[
            pl.BlockSpec((K, H, W), lambda i: (i, 0, 0)),
            pl.BlockSpec((H, W), lambda i: (0, 0)),
            pl.BlockSpec((H, W), lambda i: (0, 0)),
        ],
        out_specs=pl.BlockSpec((K, H, W), lambda i: (i, 0, 0)),
        out_shape=jax.ShapeDtypeStruct((B * C, H, W), jnp.float32),
        scratch_shapes=[pltpu.VMEM((H, W), jnp.float32)],
    )(img, weight, pruned)
    return out.reshape(1, B, C, H, W)


# speedup vs baseline: 1.1618x; 1.0005x over previous
import jax
import jax.numpy as jnp
from jax.experimental import pallas as pl
from jax.experimental.pallas import tpu as pltpu


def _apply_body(img_ref, w_ref, p_ref, out_ref):
    mw = jnp.where(p_ref[...], 0.0, w_ref[...])
    out_ref[...] = img_ref[...] * mw[None, :, :]


def kernel(image, weight, pruned):
    B, C, H, W = image.shape
    img = image.reshape(B * C, H, W)
    K = 8
    out = pl.pallas_call(
        _apply_body,
        grid=(B * C // K,),
        in_specs=[
            pl.BlockSpec((K, H, W), lambda i: (i, 0, 0)),
            pl.BlockSpec((H, W), lambda i: (0, 0)),
            pl.BlockSpec((H, W), lambda i: (0, 0)),
        ],
        out_specs=pl.BlockSpec((K, H, W), lambda i: (i, 0, 0)),
        out_shape=jax.ShapeDtypeStruct((B * C, H, W), jnp.float32),
    )(img, weight, pruned)
    return out.reshape(1, B, C, H, W)
